# baseline (speedup 1.0000x reference)
import functools

import jax
import jax.numpy as jnp
from jax import lax
from jax.experimental import pallas as pl
from jax.experimental.pallas import tpu as pltpu
from jax.experimental.pallas import tpu_sc as plsc

_NUM_FACTORS = 32
_BATCH = 16384
_NC = 2
_NS = 16
_NW = _NC * _NS
_BPW = _BATCH // _NW
_CHUNK = 128
_NCHUNK = _BPW // _CHUNK
_L = 16


def _sc_body(uid_hbm, iid_hbm, uet_hbm, ub_hbm, iet_hbm, ib_hbm, out_hbm,
             idx_u, idx_q, rows_u, rows_q, bias_u, bias_q, out_v, sem):
    wid = lax.axis_index("s") * _NC + lax.axis_index("c")
    base = wid * _BPW

    cp_u = pltpu.async_copy(uid_hbm.at[pl.ds(base, _BPW)], idx_u, sem)
    cp_q = pltpu.async_copy(iid_hbm.at[pl.ds(base, _BPW)], idx_q, sem)
    cp_u.wait()
    cp_q.wait()

    copies = []
    for c in range(_NCHUNK):
        sl = pl.ds(c * _CHUNK, _CHUNK)
        copies.append(pltpu.async_copy(ub_hbm.at[idx_u.at[sl]], bias_u.at[sl], sem))
        copies.append(pltpu.async_copy(ib_hbm.at[idx_q.at[sl]], bias_q.at[sl], sem))
        for f in range(_NUM_FACTORS):
            copies.append(pltpu.async_copy(
                uet_hbm.at[f].at[idx_u.at[sl]], rows_u.at[f].at[sl], sem))
            copies.append(pltpu.async_copy(
                iet_hbm.at[f].at[idx_q.at[sl]], rows_q.at[f].at[sl], sem))
    for cp in copies:
        cp.wait()

    def step(b, carry):
        sl = pl.ds(b * _L, _L)
        acc = bias_u[sl] + bias_q[sl]
        for f in range(_NUM_FACTORS):
            acc = acc + rows_u[f, sl] * rows_q[f, sl]
        out_v[sl] = acc
        return carry

    lax.fori_loop(0, _BPW // _L, step, 0)
    pltpu.sync_copy(out_v, out_hbm.at[pl.ds(base, _BPW)])


_mesh = plsc.VectorSubcoreMesh(core_axis_name="c", subcore_axis_name="s")

_sc_kernel = functools.partial(
    pl.kernel,
    out_type=jax.ShapeDtypeStruct((_BATCH,), jnp.float32),
    mesh=_mesh,
    compiler_params=pltpu.CompilerParams(
        needs_layout_passes=False, use_tc_tiling_on_sc=False),
    scratch_types=[
        pltpu.VMEM((_BPW,), jnp.int32),
        pltpu.VMEM((_BPW,), jnp.int32),
        pltpu.VMEM((_NUM_FACTORS, _BPW), jnp.float32),
        pltpu.VMEM((_NUM_FACTORS, _BPW), jnp.float32),
        pltpu.VMEM((_BPW,), jnp.float32),
        pltpu.VMEM((_BPW,), jnp.float32),
        pltpu.VMEM((_BPW,), jnp.float32),
        pltpu.SemaphoreType.DMA,
    ],
)(_sc_body)


def kernel(user_id, item_id, user_embeddings, user_bias, item_embeddings, item_bias):
    uid = user_id.astype(jnp.int32)
    iid = item_id.astype(jnp.int32)
    return _sc_kernel(uid, iid, user_embeddings.T, user_bias.reshape(-1),
                      item_embeddings.T, item_bias.reshape(-1))
